# baseline (speedup 1.0000x reference)
import functools

import jax
import jax.numpy as jnp
from jax import lax
from jax.experimental import pallas as pl
from jax.experimental.pallas import tpu as pltpu
from jax.experimental.pallas import tpu_sc as plsc

_LANES = 16
_UNROLL = 8


def _lane_total(v):
    lanes = lax.iota(jnp.int32, _LANES)
    dnums = lax.GatherDimensionNumbers(
        offset_dims=(), collapsed_slice_dims=(0,), start_index_map=(0,)
    )
    for sh in (8, 4, 2, 1):
        idx = lax.rem(lanes + sh, jnp.full((_LANES,), _LANES, jnp.int32))
        rot = lax.gather(
            v,
            idx[:, None],
            dnums,
            slice_sizes=(1,),
            mode=lax.GatherScatterMode.PROMISE_IN_BOUNDS,
        )
        v = v + rot
    return v[0]


def kernel(hidden_states, attention_mask):
    B, S, H = hidden_states.shape
    mesh = plsc.VectorSubcoreMesh(
        core_axis_name="c", subcore_axis_name="s", num_cores=1
    )

    @functools.partial(
        pl.kernel,
        mesh=mesh,
        out_type=jax.ShapeDtypeStruct((B, H), hidden_states.dtype),
        scratch_types=[
            pltpu.VMEM((S,), jnp.int32),
            pltpu.VMEM((H,), jnp.float32),
            pltpu.SemaphoreType.DMA,
            pltpu.SemaphoreType.DMA,
            pltpu.SemaphoreType.DMA,
            pltpu.SemaphoreType.DMA,
        ],
    )
    def _sc(hs_hbm, mask_hbm, out_hbm, mask_v, row_v, g0, g1, p0, p1):
        sid = lax.axis_index("s")

        @pl.when(sid < B)
        def _():
            b = sid
            pltpu.sync_copy(mask_hbm.at[b], mask_v)

            zero = jnp.zeros((_LANES,), jnp.int32)

            @plsc.parallel_loop(
                0, S // (_LANES * _UNROLL), carry=(zero,) * _UNROLL
            )
            def accs(i, accs):
                base = i * (_LANES * _UNROLL)
                return tuple(
                    accs[j] + mask_v[pl.ds(base + j * _LANES, _LANES)]
                    for j in range(_UNROLL)
                )
            acc = accs[0]
            for j in range(1, _UNROLL):
                acc = acc + accs[j]
            idx = _lane_total(acc) - 1
            Hh = H // 2
            c0 = pltpu.async_copy(
                hs_hbm.at[b, idx, pl.ds(0, Hh)], row_v.at[pl.ds(0, Hh)], g0
            )
            c1 = pltpu.async_copy(
                hs_hbm.at[b, idx, pl.ds(Hh, Hh)], row_v.at[pl.ds(Hh, Hh)], g1
            )
            c0.wait()
            d0 = pltpu.async_copy(
                row_v.at[pl.ds(0, Hh)], out_hbm.at[b, pl.ds(0, Hh)], p0
            )
            c1.wait()
            d1 = pltpu.async_copy(
                row_v.at[pl.ds(Hh, Hh)], out_hbm.at[b, pl.ds(Hh, Hh)], p1
            )
            d0.wait()
            d1.wait()

    return _sc(hidden_states, attention_mask)


# speedup vs baseline: 1.0001x; 1.0001x over previous
import functools

import jax
import jax.numpy as jnp
from jax import lax
from jax.experimental import pallas as pl
from jax.experimental.pallas import tpu as pltpu
from jax.experimental.pallas import tpu_sc as plsc

_LANES = 16
_UNROLL = 8


def _lane_total(v):
    lanes = lax.iota(jnp.int32, _LANES)
    dnums = lax.GatherDimensionNumbers(
        offset_dims=(), collapsed_slice_dims=(0,), start_index_map=(0,)
    )
    for sh in (8, 4, 2, 1):
        idx = lax.rem(lanes + sh, jnp.full((_LANES,), _LANES, jnp.int32))
        rot = lax.gather(
            v,
            idx[:, None],
            dnums,
            slice_sizes=(1,),
            mode=lax.GatherScatterMode.PROMISE_IN_BOUNDS,
        )
        v = v + rot
    return v[0]


def kernel(hidden_states, attention_mask):
    B, S, H = hidden_states.shape
    mesh = plsc.VectorSubcoreMesh(
        core_axis_name="c", subcore_axis_name="s", num_cores=1
    )

    @functools.partial(
        pl.kernel,
        mesh=mesh,
        out_type=jax.ShapeDtypeStruct((B, H), hidden_states.dtype),
        scratch_types=[
            pltpu.VMEM((S,), jnp.int32),
            pltpu.VMEM((H,), jnp.float32),
            pltpu.SemaphoreType.DMA,
            pltpu.SemaphoreType.DMA,
            pltpu.SemaphoreType.DMA,
            pltpu.SemaphoreType.DMA,
        ],
    )
    def _sc(hs_hbm, mask_hbm, out_hbm, mask_v, row_v, g0, g1, p0, p1):
        sid = lax.axis_index("s")

        @pl.when(sid < B)
        def _():
            b = sid
            pltpu.sync_copy(mask_hbm.at[b], mask_v)

            zero = jnp.zeros((_LANES,), jnp.int32)

            @plsc.parallel_loop(
                0, S // (_LANES * _UNROLL), carry=(zero,) * _UNROLL, unroll=2
            )
            def accs(i, accs):
                base = i * (_LANES * _UNROLL)
                return tuple(
                    accs[j] + mask_v[pl.ds(base + j * _LANES, _LANES)]
                    for j in range(_UNROLL)
                )
            acc = accs[0]
            for j in range(1, _UNROLL):
                acc = acc + accs[j]
            idx = _lane_total(acc) - 1
            Hh = H // 2
            c0 = pltpu.async_copy(
                hs_hbm.at[b, idx, pl.ds(0, Hh)], row_v.at[pl.ds(0, Hh)], g0
            )
            c1 = pltpu.async_copy(
                hs_hbm.at[b, idx, pl.ds(Hh, Hh)], row_v.at[pl.ds(Hh, Hh)], g1
            )
            c0.wait()
            d0 = pltpu.async_copy(
                row_v.at[pl.ds(0, Hh)], out_hbm.at[b, pl.ds(0, Hh)], p0
            )
            c1.wait()
            d1 = pltpu.async_copy(
                row_v.at[pl.ds(Hh, Hh)], out_hbm.at[b, pl.ds(Hh, Hh)], p1
            )
            d0.wait()
            d1.wait()

    return _sc(hidden_states, attention_mask)


# speedup vs baseline: 1.0050x; 1.0049x over previous
import functools

import jax
import jax.numpy as jnp
from jax import lax
from jax.experimental import pallas as pl
from jax.experimental.pallas import tpu as pltpu
from jax.experimental.pallas import tpu_sc as plsc

_LANES = 16
_UNROLL = 8


def _lane_total(v):
    lanes = lax.iota(jnp.int32, _LANES)
    dnums = lax.GatherDimensionNumbers(
        offset_dims=(), collapsed_slice_dims=(0,), start_index_map=(0,)
    )
    for sh in (8, 4, 2, 1):
        idx = lax.rem(lanes + sh, jnp.full((_LANES,), _LANES, jnp.int32))
        rot = lax.gather(
            v,
            idx[:, None],
            dnums,
            slice_sizes=(1,),
            mode=lax.GatherScatterMode.PROMISE_IN_BOUNDS,
        )
        v = v + rot
    return v[0]


def kernel(hidden_states, attention_mask):
    B, S, H = hidden_states.shape
    mesh = plsc.VectorSubcoreMesh(
        core_axis_name="c", subcore_axis_name="s", num_cores=1
    )

    @functools.partial(
        pl.kernel,
        mesh=mesh,
        out_type=jax.ShapeDtypeStruct((B, H), hidden_states.dtype),
        scratch_types=[
            pltpu.VMEM((S,), jnp.int32),
            pltpu.VMEM((H,), jnp.float32),
            pltpu.SemaphoreType.DMA,
            pltpu.SemaphoreType.DMA,
            pltpu.SemaphoreType.DMA,
            pltpu.SemaphoreType.DMA,
        ],
    )
    def _sc(hs_hbm, mask_hbm, out_hbm, mask_v, row_v, g0, g1, p0, p1):
        sid = lax.axis_index("s")

        @pl.when(sid < B)
        def _():
            b = sid
            pltpu.sync_copy(mask_hbm.at[b], mask_v)

            zero = jnp.zeros((_LANES,), jnp.int32)

            @plsc.parallel_loop(
                0, S // (_LANES * _UNROLL), carry=(zero,) * _UNROLL, unroll=4
            )
            def accs(i, accs):
                base = i * (_LANES * _UNROLL)
                return tuple(
                    accs[j] + mask_v[pl.ds(base + j * _LANES, _LANES)]
                    for j in range(_UNROLL)
                )
            acc = accs[0]
            for j in range(1, _UNROLL):
                acc = acc + accs[j]
            idx = _lane_total(acc) - 1
            Hh = H // 2
            c0 = pltpu.async_copy(
                hs_hbm.at[b, idx, pl.ds(0, Hh)], row_v.at[pl.ds(0, Hh)], g0
            )
            c1 = pltpu.async_copy(
                hs_hbm.at[b, idx, pl.ds(Hh, Hh)], row_v.at[pl.ds(Hh, Hh)], g1
            )
            c0.wait()
            d0 = pltpu.async_copy(
                row_v.at[pl.ds(0, Hh)], out_hbm.at[b, pl.ds(0, Hh)], p0
            )
            c1.wait()
            d1 = pltpu.async_copy(
                row_v.at[pl.ds(Hh, Hh)], out_hbm.at[b, pl.ds(Hh, Hh)], p1
            )
            d0.wait()
            d1.wait()

    return _sc(hidden_states, attention_mask)
